# Initial kernel scaffold; baseline (speedup 1.0000x reference)
import jax
import jax.numpy as jnp
from jax.experimental import pallas as pl


def kernel(position_ids, positional_encoding):
    raise NotImplementedError("write your pallas kernel here")


# speedup vs baseline: 2.3636x; 2.3636x over previous
import functools

import jax
import jax.numpy as jnp
from jax import lax
from jax.experimental import pallas as pl
from jax.experimental.pallas import tpu as pltpu
from jax.experimental.pallas import tpu_sc as plsc

D_MODEL = 1024
NUM_WORKERS = 32
CHUNK = 32


def _gather_body(b_per_w, n_chunks, ids_hbm, table_hbm, out_hbm,
                 idx_v, rows_v, sems):
    nc = 2
    wid = lax.axis_index("s") * nc + lax.axis_index("c")
    base = wid * b_per_w

    pltpu.sync_copy(ids_hbm.at[pl.ds(base, b_per_w)], idx_v)

    def _issue(c):
        buf = lax.rem(c, 2)
        pltpu.make_async_copy(
            table_hbm.at[idx_v.at[pl.ds(c * CHUNK, CHUNK)]],
            rows_v.at[buf],
            sems.at[buf],
        ).start()

    _issue(0)

    def body(c, carry):
        buf = lax.rem(c, 2)

        @pl.when(c + 1 < n_chunks)
        def _():
            _issue(c + 1)

        pltpu.make_async_copy(
            table_hbm.at[idx_v.at[pl.ds(c * CHUNK, CHUNK)]],
            rows_v.at[buf],
            sems.at[buf],
        ).wait()
        pltpu.sync_copy(rows_v.at[buf],
                        out_hbm.at[pl.ds(base + c * CHUNK, CHUNK)])
        return carry

    lax.fori_loop(0, n_chunks, body, 0)


def kernel(position_ids, positional_encoding):
    batch, seq = position_ids.shape
    n_rows = batch * seq
    b_per_w = n_rows // NUM_WORKERS
    n_chunks = b_per_w // CHUNK

    ids = position_ids.reshape(n_rows).astype(jnp.int32)

    mesh = plsc.VectorSubcoreMesh(core_axis_name="c", subcore_axis_name="s")
    body = functools.partial(_gather_body, b_per_w, n_chunks)
    out = pl.kernel(
        body,
        out_type=jax.ShapeDtypeStruct((n_rows, D_MODEL), jnp.float32),
        mesh=mesh,
        scratch_types=[
            pltpu.VMEM((b_per_w,), jnp.int32),
            pltpu.VMEM((2, CHUNK, D_MODEL), jnp.float32),
            pltpu.SemaphoreType.DMA((2,)),
        ],
    )(ids, positional_encoding)
    return out.reshape(batch, seq, D_MODEL)


# speedup vs baseline: 2.3797x; 1.0068x over previous
import functools

import jax
import jax.numpy as jnp
from jax import lax
from jax.experimental import pallas as pl
from jax.experimental.pallas import tpu as pltpu
from jax.experimental.pallas import tpu_sc as plsc

D_MODEL = 1024
NUM_WORKERS = 32
CHUNK = 32
NBUF = 3


def _gather_body(b_per_w, n_chunks, ids_hbm, table_hbm, out_hbm,
                 idx_v, rows_v, gsems, ssems):
    nc = 2
    wid = lax.axis_index("s") * nc + lax.axis_index("c")
    base = wid * b_per_w

    pltpu.sync_copy(ids_hbm.at[pl.ds(base, b_per_w)], idx_v)

    def gather(c, buf):
        return pltpu.make_async_copy(
            table_hbm.at[idx_v.at[pl.ds(c * CHUNK, CHUNK)]],
            rows_v.at[buf],
            gsems.at[buf],
        )

    def scatter(c, buf):
        return pltpu.make_async_copy(
            rows_v.at[buf],
            out_hbm.at[pl.ds(base + c * CHUNK, CHUNK)],
            ssems.at[buf],
        )

    for k in range(NBUF):
        gather(k, k).start()

    def body(c, carry):
        buf = lax.rem(c, NBUF)

        @pl.when(c >= 1)
        def _():
            pbuf = lax.rem(c - 1, NBUF)
            scatter(c - 1, pbuf).wait()

            @pl.when(c - 1 + NBUF < n_chunks)
            def _():
                gather(c - 1 + NBUF, pbuf).start()

        gather(c, buf).wait()
        scatter(c, buf).start()
        return carry

    lax.fori_loop(0, n_chunks, body, 0)
    scatter(n_chunks - 1, lax.rem(n_chunks - 1, NBUF)).wait()


def kernel(position_ids, positional_encoding):
    batch, seq = position_ids.shape
    n_rows = batch * seq
    b_per_w = n_rows // NUM_WORKERS
    n_chunks = b_per_w // CHUNK

    ids = position_ids.reshape(n_rows).astype(jnp.int32)

    mesh = plsc.VectorSubcoreMesh(core_axis_name="c", subcore_axis_name="s")
    body = functools.partial(_gather_body, b_per_w, n_chunks)
    out = pl.kernel(
        body,
        out_type=jax.ShapeDtypeStruct((n_rows, D_MODEL), jnp.float32),
        mesh=mesh,
        scratch_types=[
            pltpu.VMEM((b_per_w,), jnp.int32),
            pltpu.VMEM((NBUF, CHUNK, D_MODEL), jnp.float32),
            pltpu.SemaphoreType.DMA((NBUF,)),
            pltpu.SemaphoreType.DMA((NBUF,)),
        ],
    )(ids, positional_encoding)
    return out.reshape(batch, seq, D_MODEL)
